# Initial kernel scaffold; baseline (speedup 1.0000x reference)
import jax
import jax.numpy as jnp
from jax.experimental import pallas as pl


def kernel(embs, triplets):
    raise NotImplementedError("write your pallas kernel here")


# speedup vs baseline: 13.2242x; 13.2242x over previous
import functools

import jax
import jax.numpy as jnp
from jax import lax
from jax.experimental import pallas as pl
from jax.experimental.pallas import tpu as pltpu
from jax.experimental.pallas import tpu_sc as plsc

_GATHER_DNUMS = lax.GatherDimensionNumbers(
    offset_dims=(), collapsed_slice_dims=(0,), start_index_map=(0,))


def _lane_shuffle(x, idx16):
    return lax.gather(x, idx16[:, None], _GATHER_DNUMS, (1,),
                      mode=lax.GatherScatterMode.PROMISE_IN_BOUNDS)


N = 16384
D = 64
T = 262144
MARGIN = 1.0

NC = 2
NS = 16
NW = NC * NS
TPW = T // NW
CHUNK = 128
STEPS = TPW // CHUNK


def _make_sc_call():
    mesh = plsc.VectorSubcoreMesh(
        core_axis_name="c", subcore_axis_name="s",
        num_cores=NC, num_subcores=NS)

    @functools.partial(
        pl.kernel,
        out_type=jax.ShapeDtypeStruct((NW, 16), jnp.float32),
        mesh=mesh,
        compiler_params=pltpu.CompilerParams(use_tc_tiling_on_sc=False),
        scratch_types=[
            pltpu.VMEM((STEPS, CHUNK), jnp.int32),
            pltpu.VMEM((STEPS, CHUNK), jnp.int32),
            pltpu.VMEM((STEPS, CHUNK), jnp.int32),
            pltpu.VMEM((CHUNK, D), jnp.float32),
            pltpu.VMEM((CHUNK, D), jnp.float32),
            pltpu.VMEM((CHUNK, D), jnp.float32),
            pltpu.VMEM((16,), jnp.float32),
            pltpu.SemaphoreType.DMA,
        ],
    )
    def sc_kernel(embs_hbm, ai_hbm, pi_hbm, ni_hbm, out_hbm,
                  ai_v, pi_v, ni_v, a_r, p_r, n_r, o_v, sem):
        wid = lax.axis_index("s") * NC + lax.axis_index("c")

        pltpu.sync_copy(ai_hbm.at[wid], ai_v)
        pltpu.sync_copy(pi_hbm.at[wid], pi_v)
        pltpu.sync_copy(ni_hbm.at[wid], ni_v)

        def triplet_body(t, acc):
            s = jnp.zeros((16,), jnp.float32)
            for j in range(D // 16):
                sl = pl.ds(j * 16, 16)
                av = a_r[t, sl]
                pv = p_r[t, sl]
                nv = n_r[t, sl]
                s = s + (pv - nv) * ((pv + nv) - (av + av))
            lane = lax.iota(jnp.int32, 16)
            for k in (8, 4, 2, 1):
                s = s + _lane_shuffle(s, lane ^ k)
            return acc + jnp.maximum(s + MARGIN, 0.0)

        def step_body(step, acc):
            c1 = pltpu.async_copy(embs_hbm.at[ai_v.at[step]], a_r, sem)
            c2 = pltpu.async_copy(embs_hbm.at[pi_v.at[step]], p_r, sem)
            c3 = pltpu.async_copy(embs_hbm.at[ni_v.at[step]], n_r, sem)
            c1.wait()
            c2.wait()
            c3.wait()
            return lax.fori_loop(0, CHUNK, triplet_body, acc, unroll=4)

        acc = lax.fori_loop(0, STEPS, step_body, jnp.zeros((16,), jnp.float32))
        lane = lax.iota(jnp.int32, 16)
        o_v[...] = jnp.where(lane == 15, acc, 0.0)
        pltpu.sync_copy(o_v, out_hbm.at[wid])

    return sc_kernel


_sc_call = _make_sc_call()


@jax.jit
def kernel(embs, triplets):
    idx = triplets.T.reshape(3, NW, STEPS, CHUNK)
    partials = _sc_call(embs, idx[0], idx[1], idx[2])
    return jnp.sum(partials) / T
